# baseline (speedup 1.0000x reference)
import numpy as np
import jax
import jax.numpy as jnp
from jax.experimental import pallas as pl
from jax.experimental.pallas import tpu as pltpu

_VP_DIM = 56
_LANES = 128
_SPLITS = 2
_MAX_TILE = 3584


def _round_up(x, m):
    return ((x + m - 1) // m) * m


def _vploss_kernel(lam_ref, c0_ref, vp_ref, lab_ref, sum_ref, cnt_ref,
                   acc_sum, acc_cnt):
    step = pl.program_id(1)

    @pl.when(step == 0)
    def _init():
        acc_sum[...] = jnp.zeros_like(acc_sum)
        acc_cnt[...] = jnp.zeros_like(acc_cnt)

    vp = vp_ref[...]
    lab = lab_ref[...]
    c0 = c0_ref[...] != 0.0
    w0 = 1.0 + lam_ref[0, 0]

    s = jax.nn.sigmoid(vp)
    val = jnp.where(c0, s, vp)
    a = jnp.abs(val - lab)
    a = jnp.where(c0, a * w0, a)
    m = lab != 0.0

    acc_sum[...] += jnp.where(m, a, 0.0)
    acc_cnt[...] += jnp.where(m, 1.0, 0.0)

    @pl.when(step == pl.num_programs(1) - 1)
    def _finalize():
        sum_ref[...] = jnp.full((1, 8, _LANES), jnp.sum(acc_sum[...]),
                                jnp.float32)
        cnt_ref[...] = jnp.full((1, 8, _LANES), jnp.sum(acc_cnt[...]),
                                jnp.float32)


def kernel(vp, vp_label, lambda_vp):
    n = vp.size
    assert vp.shape == vp_label.shape and vp.shape[-1] == _VP_DIM

    vpf = vp.reshape(-1).astype(jnp.float32)
    labf = vp_label.reshape(-1).astype(jnp.float32)

    rows = -(-n // _LANES)
    tile = min(_MAX_TILE, _round_up(rows, 56))
    rows_pad = _round_up(rows, tile * _SPLITS)
    if rows_pad * _LANES != n:
        pad = rows_pad * _LANES - n
        vpf = jnp.pad(vpf, (0, pad))
        labf = jnp.pad(labf, (0, pad))
    vp2 = vpf.reshape(rows_pad, _LANES)
    lab2 = labf.reshape(rows_pad, _LANES)
    tiles_per_split = rows_pad // (tile * _SPLITS)

    base = (np.arange(7 * _LANES) % _VP_DIM == 0).reshape(7, _LANES)
    c0_pat = jnp.asarray(np.tile(base, (tile // 7, 1)), jnp.float32)

    lam = jnp.asarray(lambda_vp, jnp.float32).reshape(1, 1)

    itemsize = jnp.dtype(vp.dtype).itemsize
    cost = pl.CostEstimate(
        flops=10 * n,
        transcendentals=n,
        bytes_accessed=2 * n * itemsize,
    )

    sums, cnts = pl.pallas_call(
        _vploss_kernel,
        out_shape=(
            jax.ShapeDtypeStruct((_SPLITS, 8, _LANES), jnp.float32),
            jax.ShapeDtypeStruct((_SPLITS, 8, _LANES), jnp.float32),
        ),
        grid=(_SPLITS, tiles_per_split),
        in_specs=[
            pl.BlockSpec(memory_space=pltpu.SMEM),
            pl.BlockSpec((tile, _LANES), lambda c, i: (0, 0)),
            pl.BlockSpec((tile, _LANES),
                         lambda c, i: (c * tiles_per_split + i, 0)),
            pl.BlockSpec((tile, _LANES),
                         lambda c, i: (c * tiles_per_split + i, 0)),
        ],
        out_specs=(
            pl.BlockSpec((1, 8, _LANES), lambda c, i: (c, 0, 0)),
            pl.BlockSpec((1, 8, _LANES), lambda c, i: (c, 0, 0)),
        ),
        scratch_shapes=[
            pltpu.VMEM((tile, _LANES), jnp.float32),
            pltpu.VMEM((tile, _LANES), jnp.float32),
        ],
        compiler_params=pltpu.CompilerParams(
            dimension_semantics=("parallel", "arbitrary"),
            vmem_limit_bytes=64 * 1024 * 1024,
        ),
        cost_estimate=cost,
    )(lam, c0_pat, vp2, lab2)

    total = jnp.sum(sums[:, 0, 0])
    count = jnp.sum(cnts[:, 0, 0])
    return total / count


# speedup vs baseline: 7.6148x; 7.6148x over previous
import jax
import jax.numpy as jnp
from jax.experimental import pallas as pl
from jax.experimental.pallas import tpu as pltpu

_VP_DIM = 56
_LANES = 128


def _round_up(x, m):
    return ((x + m - 1) // m) * m


def _vploss_kernel(lam_ref, vp_ref, lab_ref, sum_ref, cnt_ref,
                   acc_sum, acc_cnt, acc_cor):
    step = pl.program_id(1)

    @pl.when(step == 0)
    def _init():
        acc_sum[...] = jnp.zeros_like(acc_sum)
        acc_cnt[...] = jnp.zeros_like(acc_cnt)
        acc_cor[...] = jnp.zeros_like(acc_cor)

    vp = vp_ref[...]
    lab = lab_ref[...]
    m = lab != 0.0
    a = jnp.abs(vp - lab)
    acc_sum[...] += jnp.where(m, a, 0.0)
    acc_cnt[...] += jnp.where(m, 1.0, 0.0)

    vp0 = vp_ref[:, 0:1, :]
    lab0 = lab_ref[:, 0:1, :]
    w0 = 1.0 + lam_ref[0, 0]
    s0 = jax.nn.sigmoid(vp0)
    m0 = lab0 != 0.0
    delta = w0 * jnp.abs(s0 - lab0) - jnp.abs(vp0 - lab0)
    acc_cor[...] += jnp.where(m0, delta, 0.0)

    @pl.when(step == pl.num_programs(1) - 1)
    def _finalize():
        total = jnp.sum(acc_sum[...]) + jnp.sum(acc_cor[...])
        sum_ref[...] = jnp.full((1, 8, _LANES), total, jnp.float32)
        cnt_ref[...] = jnp.full((1, 8, _LANES), jnp.sum(acc_cnt[...]),
                                jnp.float32)


def kernel(vp, vp_label, lambda_vp):
    assert vp.shape == vp_label.shape and vp.shape[-1] == _VP_DIM

    if vp.ndim == 3:
        vpt = jnp.transpose(vp, (1, 2, 0)).astype(jnp.float32)
        labt = jnp.transpose(vp_label, (1, 2, 0)).astype(jnp.float32)
    else:
        n = 1
        for d in vp.shape[:-1]:
            n *= d
        vpt = jnp.transpose(vp.reshape(n, _VP_DIM)).reshape(
            1, _VP_DIM, n).astype(jnp.float32)
        labt = jnp.transpose(vp_label.reshape(n, _VP_DIM)).reshape(
            1, _VP_DIM, n).astype(jnp.float32)

    v, _, n = vpt.shape
    n_pad = _round_up(n, _LANES)
    splits = 2 if v % 2 == 0 else 1
    blk = 1
    for cand in (4, 2, 1):
        if (v // splits) % cand == 0:
            blk = cand
            break
    v_pad = _round_up(v, splits * blk)
    if n_pad != n or v_pad != v:
        pad = ((0, v_pad - v), (0, 0), (0, n_pad - n))
        vpt = jnp.pad(vpt, pad)
        labt = jnp.pad(labt, pad)
    tiles_per_split = v_pad // (blk * splits)

    lam = jnp.asarray(lambda_vp, jnp.float32).reshape(1, 1)

    nelem = v_pad * _VP_DIM * n_pad
    cost = pl.CostEstimate(
        flops=8 * nelem,
        transcendentals=nelem // _VP_DIM,
        bytes_accessed=2 * 4 * nelem,
    )

    sums, cnts = pl.pallas_call(
        _vploss_kernel,
        out_shape=(
            jax.ShapeDtypeStruct((splits, 8, _LANES), jnp.float32),
            jax.ShapeDtypeStruct((splits, 8, _LANES), jnp.float32),
        ),
        grid=(splits, tiles_per_split),
        in_specs=[
            pl.BlockSpec(memory_space=pltpu.SMEM),
            pl.BlockSpec((blk, _VP_DIM, n_pad),
                         lambda c, i: (c * tiles_per_split + i, 0, 0)),
            pl.BlockSpec((blk, _VP_DIM, n_pad),
                         lambda c, i: (c * tiles_per_split + i, 0, 0)),
        ],
        out_specs=(
            pl.BlockSpec((1, 8, _LANES), lambda c, i: (c, 0, 0)),
            pl.BlockSpec((1, 8, _LANES), lambda c, i: (c, 0, 0)),
        ),
        scratch_shapes=[
            pltpu.VMEM((blk, _VP_DIM, n_pad), jnp.float32),
            pltpu.VMEM((blk, _VP_DIM, n_pad), jnp.float32),
            pltpu.VMEM((blk, 1, n_pad), jnp.float32),
        ],
        compiler_params=pltpu.CompilerParams(
            dimension_semantics=("parallel", "arbitrary"),
        ),
        cost_estimate=cost,
    )(lam, vpt, labt)

    total = jnp.sum(sums[:, 0, 0])
    count = jnp.sum(cnts[:, 0, 0])
    return total / count


# speedup vs baseline: 8.7451x; 1.1484x over previous
import jax
import jax.numpy as jnp
from jax.experimental import pallas as pl
from jax.experimental.pallas import tpu as pltpu

_VP_DIM = 56
_LANES = 128


def _round_up(x, m):
    return ((x + m - 1) // m) * m


def _vploss_kernel(lam_ref, vp_ref, lab_ref, out_ref,
                   acc_sum, acc_cnt, acc_cor):
    step = pl.program_id(0)

    @pl.when(step == 0)
    def _init():
        acc_sum[...] = jnp.zeros_like(acc_sum)
        acc_cnt[...] = jnp.zeros_like(acc_cnt)
        acc_cor[...] = jnp.zeros_like(acc_cor)

    vp = vp_ref[...]
    lab = lab_ref[...]
    m = lab != 0.0
    a = jnp.abs(vp - lab)
    acc_sum[...] += jnp.where(m, a, 0.0)
    acc_cnt[...] += jnp.where(m, 1.0, 0.0)

    vp0 = vp_ref[:, 0:1, :]
    lab0 = lab_ref[:, 0:1, :]
    w0 = 1.0 + lam_ref[0, 0]
    s0 = jax.nn.sigmoid(vp0)
    m0 = lab0 != 0.0
    delta = w0 * jnp.abs(s0 - lab0) - jnp.abs(vp0 - lab0)
    acc_cor[...] += jnp.where(m0, delta, 0.0)

    @pl.when(step == pl.num_programs(0) - 1)
    def _finalize():
        total = jnp.sum(acc_sum[...]) + jnp.sum(acc_cor[...])
        count = jnp.sum(acc_cnt[...])
        out_ref[0, 0] = total / count


def kernel(vp, vp_label, lambda_vp):
    assert vp.shape == vp_label.shape and vp.shape[-1] == _VP_DIM

    if vp.ndim == 3:
        vpt = jnp.transpose(vp, (1, 2, 0)).astype(jnp.float32)
        labt = jnp.transpose(vp_label, (1, 2, 0)).astype(jnp.float32)
    else:
        n = 1
        for d in vp.shape[:-1]:
            n *= d
        vpt = jnp.transpose(vp.reshape(n, _VP_DIM)).reshape(
            1, _VP_DIM, n).astype(jnp.float32)
        labt = jnp.transpose(vp_label.reshape(n, _VP_DIM)).reshape(
            1, _VP_DIM, n).astype(jnp.float32)

    v, _, n = vpt.shape
    n_pad = _round_up(n, _LANES)
    blk = 1
    for cand in (4, 2, 1):
        if v % cand == 0:
            blk = cand
            break
    v_pad = _round_up(v, blk)
    if n_pad != n or v_pad != v:
        pad = ((0, v_pad - v), (0, 0), (0, n_pad - n))
        vpt = jnp.pad(vpt, pad)
        labt = jnp.pad(labt, pad)
    steps = v_pad // blk

    lam = jnp.asarray(lambda_vp, jnp.float32).reshape(1, 1)

    nelem = v_pad * _VP_DIM * n_pad
    cost = pl.CostEstimate(
        flops=8 * nelem,
        transcendentals=nelem // _VP_DIM,
        bytes_accessed=2 * 4 * nelem,
    )

    out = pl.pallas_call(
        _vploss_kernel,
        out_shape=jax.ShapeDtypeStruct((1, 1), jnp.float32),
        grid=(steps,),
        in_specs=[
            pl.BlockSpec(memory_space=pltpu.SMEM),
            pl.BlockSpec((blk, _VP_DIM, n_pad), lambda i: (i, 0, 0)),
            pl.BlockSpec((blk, _VP_DIM, n_pad), lambda i: (i, 0, 0)),
        ],
        out_specs=pl.BlockSpec(memory_space=pltpu.SMEM),
        scratch_shapes=[
            pltpu.VMEM((blk, _VP_DIM, n_pad), jnp.float32),
            pltpu.VMEM((blk, _VP_DIM, n_pad), jnp.float32),
            pltpu.VMEM((blk, 1, n_pad), jnp.float32),
        ],
        compiler_params=pltpu.CompilerParams(
            dimension_semantics=("arbitrary",),
        ),
        cost_estimate=cost,
    )(lam, vpt, labt)

    return out.reshape(())


# speedup vs baseline: 9.0609x; 1.0361x over previous
import jax
import jax.numpy as jnp
from jax.experimental import pallas as pl
from jax.experimental.pallas import tpu as pltpu

_VP_DIM = 56
_LANES = 128


def _round_up(x, m):
    return ((x + m - 1) // m) * m


def _vploss_kernel(lam_ref, vp_ref, lab_ref, out_ref,
                   acc_sum, acc_cnt, acc_cor):
    step = pl.program_id(0)
    blk, dim, n = vp_ref.shape

    @pl.when(step == 0)
    def _init():
        acc_sum[...] = jnp.zeros_like(acc_sum)
        acc_cnt[...] = jnp.zeros_like(acc_cnt)
        acc_cor[...] = jnp.zeros_like(acc_cor)

    vp = vp_ref[...]
    lab = lab_ref[...]
    m = lab != 0.0
    a = jnp.abs(vp - lab)
    am = jnp.where(m, a, 0.0).reshape(blk * dim // 8, 8, n)
    mf = jnp.where(m, 1.0, 0.0).reshape(blk * dim // 8, 8, n)
    acc_sum[...] += jnp.sum(am, axis=0)
    acc_cnt[...] += jnp.sum(mf, axis=0)

    vp0 = vp_ref[:, 0:1, :]
    lab0 = lab_ref[:, 0:1, :]
    w0 = 1.0 + lam_ref[0, 0]
    s0 = jax.nn.sigmoid(vp0)
    m0 = lab0 != 0.0
    delta = w0 * jnp.abs(s0 - lab0) - jnp.abs(vp0 - lab0)
    acc_cor[...] += jnp.where(m0, delta, 0.0)

    @pl.when(step == pl.num_programs(0) - 1)
    def _finalize():
        total = jnp.sum(acc_sum[...]) + jnp.sum(acc_cor[...])
        count = jnp.sum(acc_cnt[...])
        out_ref[0, 0] = total / count


def kernel(vp, vp_label, lambda_vp):
    assert vp.shape == vp_label.shape and vp.shape[-1] == _VP_DIM

    if vp.ndim == 3:
        vpt = jnp.transpose(vp, (1, 2, 0)).astype(jnp.float32)
        labt = jnp.transpose(vp_label, (1, 2, 0)).astype(jnp.float32)
    else:
        n = 1
        for d in vp.shape[:-1]:
            n *= d
        vpt = jnp.transpose(vp.reshape(n, _VP_DIM)).reshape(
            1, _VP_DIM, n).astype(jnp.float32)
        labt = jnp.transpose(vp_label.reshape(n, _VP_DIM)).reshape(
            1, _VP_DIM, n).astype(jnp.float32)

    v, _, n = vpt.shape
    n_pad = _round_up(n, _LANES)
    blk = 1
    for cand in (4, 2, 1):
        if v % cand == 0:
            blk = cand
            break
    v_pad = _round_up(v, blk)
    if n_pad != n or v_pad != v:
        pad = ((0, v_pad - v), (0, 0), (0, n_pad - n))
        vpt = jnp.pad(vpt, pad)
        labt = jnp.pad(labt, pad)
    steps = v_pad // blk

    lam = jnp.asarray(lambda_vp, jnp.float32).reshape(1, 1)

    nelem = v_pad * _VP_DIM * n_pad
    cost = pl.CostEstimate(
        flops=8 * nelem,
        transcendentals=nelem // _VP_DIM,
        bytes_accessed=2 * 4 * nelem,
    )

    out = pl.pallas_call(
        _vploss_kernel,
        out_shape=jax.ShapeDtypeStruct((1, 1), jnp.float32),
        grid=(steps,),
        in_specs=[
            pl.BlockSpec(memory_space=pltpu.SMEM),
            pl.BlockSpec((blk, _VP_DIM, n_pad), lambda i: (i, 0, 0)),
            pl.BlockSpec((blk, _VP_DIM, n_pad), lambda i: (i, 0, 0)),
        ],
        out_specs=pl.BlockSpec(memory_space=pltpu.SMEM),
        scratch_shapes=[
            pltpu.VMEM((8, n_pad), jnp.float32),
            pltpu.VMEM((8, n_pad), jnp.float32),
            pltpu.VMEM((blk, 1, n_pad), jnp.float32),
        ],
        compiler_params=pltpu.CompilerParams(
            dimension_semantics=("arbitrary",),
        ),
        cost_estimate=cost,
    )(lam, vpt, labt)

    return out.reshape(())
